# baseline (speedup 1.0000x reference)
import functools

import jax
import jax.numpy as jnp
from jax import lax
from jax.experimental import pallas as pl
from jax.experimental.pallas import tpu as pltpu
from jax.experimental.pallas import tpu_sc as plsc

L = 16


def _make_sc_kernel(B, E, D, V):
    NW = 32
    SPAN = E // NW
    CP = 16
    NCH = SPAN // CP
    SL = D // L

    mesh = plsc.VectorSubcoreMesh(core_axis_name="c", subcore_axis_name="s")

    @functools.partial(
        pl.kernel,
        out_type=jax.ShapeDtypeStruct((B * E, D), jnp.float32),
        mesh=mesh,
        scratch_types=[
            pltpu.VMEM((B, SPAN), jnp.int32),
            pltpu.VMEM((CP, D), jnp.float32),
            pltpu.VMEM((CP, D), jnp.float32),
            pltpu.SemaphoreType.DMA,
        ],
    )
    def k(table_hbm, ids_hbm, pos_hbm, out_hbm, idx_v, pos_v, rows_v, sem):
        wid = lax.axis_index("s") * 2 + lax.axis_index("c")
        p0 = wid * SPAN

        for b in range(B):
            pltpu.sync_copy(ids_hbm.at[pl.ds(b * E + p0, SPAN)], idx_v.at[b])

        def chunk(pc, _):
            e0 = p0 + pc * CP
            pltpu.sync_copy(pos_hbm.at[pl.ds(e0, CP)], pos_v)
            for b in range(B):
                pltpu.async_copy(
                    table_hbm.at[idx_v.at[b, pl.ds(pc * CP, CP)]],
                    rows_v, sem,
                ).wait()

                def add_row(r, _):
                    for s in range(SL):
                        plsc.addupdate(
                            rows_v.at[r, pl.ds(s * L, L)],
                            pos_v[r, pl.ds(s * L, L)],
                        )
                    return 0
                lax.fori_loop(0, CP, add_row, 0)

                pltpu.sync_copy(rows_v, out_hbm.at[pl.ds(b * E + e0, CP)])
            return 0

        lax.fori_loop(0, NCH, chunk, 0)

    return k


def kernel(input_ids, patch_table, position_embeddings):
    B, E = input_ids.shape
    V, D = patch_table.shape
    ids_flat = input_ids.reshape(B * E).astype(jnp.int32)
    pos2d = position_embeddings.reshape(E, D)
    k = _make_sc_kernel(B, E, D, V)
    out2d = k(patch_table, ids_flat, pos2d)
    return out2d.reshape(B, E, D)


# speedup vs baseline: 1.0396x; 1.0396x over previous
import functools

import jax
import jax.numpy as jnp
from jax import lax
from jax.experimental import pallas as pl
from jax.experimental.pallas import tpu as pltpu
from jax.experimental.pallas import tpu_sc as plsc

L = 16


def _make_sc_kernel(B, E, D, V):
    NW = 32
    SPAN = E // NW
    CP = 8
    NCH = SPAN // CP
    SL = D // L

    mesh = plsc.VectorSubcoreMesh(core_axis_name="c", subcore_axis_name="s")

    @functools.partial(
        pl.kernel,
        out_type=jax.ShapeDtypeStruct((B * E, D), jnp.float32),
        mesh=mesh,
        scratch_types=[
            pltpu.VMEM((B, SPAN), jnp.int32),
            pltpu.VMEM((2, CP, D), jnp.float32),
            pltpu.VMEM((B, 2, CP, D), jnp.float32),
            pltpu.SemaphoreType.DMA((2,)),
            pltpu.SemaphoreType.DMA((B, 2)),
            pltpu.SemaphoreType.DMA((B, 2)),
        ],
    )
    def k(table_hbm, ids_hbm, pos_hbm, out_hbm,
          idx_v, pos_v, rows_v, psem, gsem, wsem):
        wid = lax.axis_index("s") * 2 + lax.axis_index("c")
        p0 = wid * SPAN

        for b in range(B):
            pltpu.sync_copy(ids_hbm.at[pl.ds(b * E + p0, SPAN)], idx_v.at[b])

        def issue_pos(pc, par):
            pltpu.async_copy(pos_hbm.at[pl.ds(p0 + pc * CP, CP)],
                             pos_v.at[par], psem.at[par])

        def issue_gather(pc, b, par):
            pltpu.async_copy(table_hbm.at[idx_v.at[b, pl.ds(pc * CP, CP)]],
                             rows_v.at[b, par], gsem.at[b, par])

        issue_pos(0, 0)
        for b in range(B):
            issue_gather(0, b, 0)

        def chunk(pc, _):
            par = lax.rem(pc, 2)
            nxt = 1 - par
            @pl.when(pc < NCH - 1)
            def _():
                issue_pos(pc + 1, nxt)
            pltpu.make_async_copy(pos_hbm.at[pl.ds(0, CP)],
                                  pos_v.at[par], psem.at[par]).wait()
            for b in range(B):
                pltpu.make_async_copy(
                    table_hbm.at[idx_v.at[b, pl.ds(0, CP)]],
                    rows_v.at[b, par], gsem.at[b, par]).wait()

                def add_row(r, _):
                    for s in range(SL):
                        plsc.addupdate(
                            rows_v.at[b, par, r, pl.ds(s * L, L)],
                            pos_v[par, r, pl.ds(s * L, L)],
                        )
                    return 0
                lax.fori_loop(0, CP, add_row, 0, unroll=2)

                pltpu.async_copy(
                    rows_v.at[b, par],
                    out_hbm.at[pl.ds(b * E + p0 + pc * CP, CP)],
                    wsem.at[b, par])

                @pl.when(pc < NCH - 1)
                def _():
                    @pl.when(pc >= 1)
                    def _():
                        pltpu.make_async_copy(
                            rows_v.at[b, nxt],
                            out_hbm.at[pl.ds(0, CP)],
                            wsem.at[b, nxt]).wait()
                    issue_gather(pc + 1, b, nxt)
            return 0

        lax.fori_loop(0, NCH, chunk, 0)

        last = (NCH - 1) % 2
        for b in range(B):
            for par in (last, 1 - last):
                pltpu.make_async_copy(
                    rows_v.at[b, par],
                    out_hbm.at[pl.ds(0, CP)],
                    wsem.at[b, par]).wait()

    return k


def kernel(input_ids, patch_table, position_embeddings):
    B, E = input_ids.shape
    V, D = patch_table.shape
    ids_flat = input_ids.reshape(B * E).astype(jnp.int32)
    pos2d = position_embeddings.reshape(E, D)
    k = _make_sc_kernel(B, E, D, V)
    out2d = k(patch_table, ids_flat, pos2d)
    return out2d.reshape(B, E, D)
